# baseline (speedup 1.0000x reference)
import functools

import jax
import jax.numpy as jnp
from jax import lax
from jax.experimental import pallas as pl
from jax.experimental.pallas import tpu as pltpu
from jax.experimental.pallas import tpu_sc as plsc

N = 10000
D = 128
R = 4
CHUNK = 64
NCHUNK = D // CHUNK
NC, NS = 2, 16
BI = 128
KB = 2
EB = KB * BI
NBLK = 4
BR = NBLK * KB
BN_EPS = 1e-5
RN = R * N
RD = 200
NRD = RN // RD
XS = N // NS


def _sc_body(x_all, idx_all, zeros, out, acc, x_s, ib0, ib1, rowsA, rowsB,
             semgA, semgB, semsA, semsB, semi0, semi1):
    core = lax.axis_index("c")
    sub = lax.axis_index("s")
    n_rows = idx_all.shape[0] // NS
    rbase = sub * n_rows

    chunk = core
    pltpu.sync_copy(x_all.at[pl.ds(chunk * N + sub * XS, XS)],
                    x_s.at[pl.ds(sub * XS, XS)])
    pltpu.sync_copy(zeros, rowsA.at[pl.ds(0, RD)])
    for k in range(-(-NRD // NS)):
        cid = sub + NS * k

        @pl.when(cid < NRD)
        def _():
            pltpu.sync_copy(rowsA.at[pl.ds(0, RD)], acc.at[pl.ds(cid * RD, RD)])

    plsc.subcore_barrier()

    U2 = n_rows // (2 * BR)
    wi0 = pltpu.make_async_copy(idx_all.at[pl.ds(rbase, BR)], ib0, semi0)
    wi1 = pltpu.make_async_copy(idx_all.at[pl.ds(rbase, BR)], ib1, semi1)
    wi0.start()
    pltpu.async_copy(idx_all.at[pl.ds(rbase + BR, BR)], ib1, semi1)

    def run_body(ib, base_done):
        bufs = [(rowsA, semgA, semsA), (rowsB, semgB, semsB)]
        g = [None] * NBLK
        s = [None] * NBLK

        def start_g(q):
            buf, semg, _ = bufs[q % 2]
            g[q] = [pltpu.async_copy(
                x_s.at[ib.at[q * KB + j, 0]], buf.at[pl.ds(j * BI, BI)], semg)
                for j in range(KB)]

        def start_s(q):
            buf, _, sems = bufs[q % 2]
            s[q] = [pltpu.async_copy(
                buf.at[pl.ds(j * BI, BI)], acc.at[ib.at[q * KB + j, 1]], sems,
                add=True) for j in range(KB)]

        start_g(0)
        start_g(1)
        for cp in g[0]:
            cp.wait()
        start_s(0)
        for cp in g[1]:
            cp.wait()
        start_s(1)
        for cp in s[0]:
            cp.wait()
        start_g(2)
        for cp in s[1]:
            cp.wait()
        start_g(3)
        for cp in g[2]:
            cp.wait()
        start_s(2)
        for cp in g[3]:
            cp.wait()
        start_s(3)
        for cp in s[2]:
            cp.wait()
        for cp in s[3]:
            cp.wait()

    def dbl(u, carry):
        wi0.wait()
        run_body(ib0, None)

        @pl.when(u < U2 - 1)
        def _():
            pltpu.async_copy(
                idx_all.at[pl.ds(rbase + (2 * u + 2) * BR, BR)], ib0, semi0)

        wi1.wait()
        run_body(ib1, None)

        @pl.when(u < U2 - 1)
        def _():
            pltpu.async_copy(
                idx_all.at[pl.ds(rbase + (2 * u + 3) * BR, BR)], ib1, semi1)

        return carry

    lax.fori_loop(0, U2, dbl, 0)
    plsc.subcore_barrier()
    for k in range(-(-NRD // NS)):
        cid = sub + NS * k

        @pl.when(cid < NRD)
        def _():
            r0 = cid * RD
            pltpu.sync_copy(acc.at[pl.ds(r0, RD)], rowsA.at[pl.ds(0, RD)])
            pltpu.sync_copy(rowsA.at[pl.ds(0, RD)], out.at[chunk, pl.ds(r0, RD)])


def _sc_aggregate(x_all, idx_all, zeros):
    mesh = plsc.VectorSubcoreMesh(
        core_axis_name="c", subcore_axis_name="s", num_cores=NC,
        num_subcores=NS)
    f = pl.kernel(
        _sc_body,
        out_type=jax.ShapeDtypeStruct((NCHUNK, RN, CHUNK), jnp.bfloat16),
        mesh=mesh,
        scratch_types=[
            pltpu.VMEM_SHARED((RN + 8, CHUNK), jnp.bfloat16),
            pltpu.VMEM_SHARED((N, CHUNK), jnp.bfloat16),
            pltpu.VMEM((BR, 2, BI), jnp.int32),
            pltpu.VMEM((BR, 2, BI), jnp.int32),
            pltpu.VMEM((EB, CHUNK), jnp.bfloat16),
            pltpu.VMEM((EB, CHUNK), jnp.bfloat16),
            pltpu.SemaphoreType.DMA,
            pltpu.SemaphoreType.DMA,
            pltpu.SemaphoreType.DMA,
            pltpu.SemaphoreType.DMA,
            pltpu.SemaphoreType.DMA,
            pltpu.SemaphoreType.DMA,
        ],
        compiler_params=pltpu.CompilerParams(use_tc_tiling_on_sc=False),
    )
    return f(x_all, idx_all, zeros)


def _mlp1_body(x_ref, agg_ref, w_ref, b_ref, h1_ref, s1_ref, s2_ref):
    nb = pl.program_id(1)
    h = x_ref[...] + jnp.concatenate(
        [agg_ref[c, 0] for c in range(NCHUNK)], axis=1).astype(jnp.float32)
    h1 = jnp.dot(h, w_ref[0], precision=lax.Precision.DEFAULT) + b_ref[0, 0]
    h1_ref[0] = h1

    @pl.when(nb == 0)
    def _():
        s1_ref[...] = jnp.zeros_like(s1_ref)
        s2_ref[...] = jnp.zeros_like(s2_ref)

    s1_ref[0] += jnp.sum(h1, axis=0, keepdims=True)
    s2_ref[0] += jnp.sum(h1 * h1, axis=0, keepdims=True)


def _mlp2_body(x_ref, h1_ref, wst_ref, w2t_ref, a_ref, c_ref, bias_ref, o_ref):
    acc = jnp.dot(x_ref[...], wst_ref[...],
                  precision=lax.Precision.DEFAULT) + bias_ref[...]
    for r in range(R):
        g = jnp.maximum(h1_ref[r] * a_ref[r] + c_ref[r], 0.0)
        acc += jnp.dot(g, w2t_ref[r], precision=lax.Precision.DEFAULT)
    o_ref[...] = acc


def kernel(x, edge_index, edge_type, W_self, b_self, W1, b1, gamma, beta,
           W2, b2):
    E = edge_index.shape[1]
    row = edge_index[0].astype(jnp.int32)
    col = edge_index[1].astype(jnp.int32)
    et = edge_type.astype(jnp.int32)

    block = NS * 2 * BR * BI
    e_pad = ((E + block - 1) // block) * block
    pad = e_pad - E
    col_p = jnp.concatenate([col, jnp.zeros((pad,), jnp.int32)])
    d_p = jnp.concatenate([et * N + row, jnp.full((pad,), RN, jnp.int32)])
    idx_all = jnp.concatenate([col_p.reshape(e_pad // BI, 1, BI),
                               d_p.reshape(e_pad // BI, 1, BI)], axis=1)
    x_all = x.astype(jnp.bfloat16).reshape(N, NCHUNK, CHUNK).transpose(
        1, 0, 2).reshape(NCHUNK * N, CHUNK)
    zeros = jnp.zeros((RD, CHUNK), jnp.bfloat16)

    aggc = _sc_aggregate(x_all, idx_all, zeros)
    agg = aggc.reshape(NCHUNK, R, N, CHUNK)

    BN = 1000
    nb_grid = N // BN
    W1T = W1.transpose(0, 2, 1)
    h1, s1, s2 = pl.pallas_call(
        _mlp1_body,
        grid=(R, nb_grid),
        in_specs=[
            pl.BlockSpec((BN, D), lambda r, nb: (nb, 0)),
            pl.BlockSpec((NCHUNK, 1, BN, CHUNK), lambda r, nb: (0, r, nb, 0)),
            pl.BlockSpec((1, D, D), lambda r, nb: (r, 0, 0)),
            pl.BlockSpec((1, 1, D), lambda r, nb: (r, 0, 0)),
        ],
        out_specs=[
            pl.BlockSpec((1, BN, D), lambda r, nb: (r, nb, 0)),
            pl.BlockSpec((1, 1, D), lambda r, nb: (r, 0, 0)),
            pl.BlockSpec((1, 1, D), lambda r, nb: (r, 0, 0)),
        ],
        out_shape=[
            jax.ShapeDtypeStruct((R, N, D), jnp.float32),
            jax.ShapeDtypeStruct((R, 1, D), jnp.float32),
            jax.ShapeDtypeStruct((R, 1, D), jnp.float32),
        ],
    )(x, agg, W1T, b1.reshape(R, 1, D))

    s1 = s1.reshape(R, D)
    s2 = s2.reshape(R, D)
    mean = s1 / N
    var = s2 / N - mean * mean
    a = gamma * lax.rsqrt(var + BN_EPS)
    c = beta - mean * a
    bias_total = (b_self + jnp.sum(b2, axis=0))[None, :]

    out = pl.pallas_call(
        _mlp2_body,
        grid=(nb_grid,),
        in_specs=[
            pl.BlockSpec((BN, D), lambda nb: (nb, 0)),
            pl.BlockSpec((R, BN, D), lambda nb: (0, nb, 0)),
            pl.BlockSpec((D, D), lambda nb: (0, 0)),
            pl.BlockSpec((R, D, D), lambda nb: (0, 0, 0)),
            pl.BlockSpec((R, D), lambda nb: (0, 0)),
            pl.BlockSpec((R, D), lambda nb: (0, 0)),
            pl.BlockSpec((1, D), lambda nb: (0, 0)),
        ],
        out_specs=pl.BlockSpec((BN, D), lambda nb: (nb, 0)),
        out_shape=jax.ShapeDtypeStruct((N, D), jnp.float32),
    )(x, h1, W_self.T, W2.transpose(0, 2, 1), a, c, bias_total)
    return out


# speedup vs baseline: 1.0464x; 1.0464x over previous
import functools

import jax
import jax.numpy as jnp
from jax import lax
from jax.experimental import pallas as pl
from jax.experimental.pallas import tpu as pltpu
from jax.experimental.pallas import tpu_sc as plsc

N = 10000
D = 128
R = 4
CHUNK = 64
NCHUNK = D // CHUNK
NC, NS = 2, 16
BI = 128
KB = 3
EB = KB * BI
BN_EPS = 1e-5
RN = R * N
RD = 200
NRD = RN // RD
XS = N // NS


def _sc_body(x_all, idx_all, zeros, out, acc, x_s, iA, iB, rowsA, rowsB,
             semgA, semgB, semsA, semsB):
    core = lax.axis_index("c")
    sub = lax.axis_index("s")
    n_rows = idx_all.shape[0] // NS
    rbase = sub * n_rows

    chunk = core
    pltpu.sync_copy(x_all.at[pl.ds(chunk * N + sub * XS, XS)],
                    x_s.at[pl.ds(sub * XS, XS)])
    pltpu.sync_copy(zeros, rowsA.at[pl.ds(0, RD)])
    for k in range(-(-NRD // NS)):
        cid = sub + NS * k

        @pl.when(cid < NRD)
        def _():
            pltpu.sync_copy(rowsA.at[pl.ds(0, RD)], acc.at[pl.ds(cid * RD, RD)])

    plsc.subcore_barrier()

    gA = [pltpu.make_async_copy(
        x_s.at[iA.at[j, 0]], rowsA.at[pl.ds(j * BI, BI)], semgA)
        for j in range(KB)]
    gB = [pltpu.make_async_copy(
        x_s.at[iB.at[j, 0]], rowsB.at[pl.ds(j * BI, BI)], semgB)
        for j in range(KB)]
    H = n_rows // (2 * KB)

    pltpu.sync_copy(idx_all.at[pl.ds(rbase, KB)], iA)
    for cp in gA:
        cp.start()

    def body(h, carry):
        pltpu.sync_copy(idx_all.at[pl.ds(rbase + (2 * h + 1) * KB, KB)], iB)
        for cp in gB:
            cp.start()
        sA = []
        for j in range(KB):
            gA[j].wait()
            sA.append(pltpu.async_copy(
                rowsA.at[pl.ds(j * BI, BI)], acc.at[iA.at[j, 1]], semsA,
                add=True))
        for cp in sA:
            cp.wait()

        @pl.when(h < H - 1)
        def _():
            pltpu.sync_copy(
                idx_all.at[pl.ds(rbase + (2 * h + 2) * KB, KB)], iA)
            for cp in gA:
                cp.start()

        sB = []
        for j in range(KB):
            gB[j].wait()
            sB.append(pltpu.async_copy(
                rowsB.at[pl.ds(j * BI, BI)], acc.at[iB.at[j, 1]], semsB,
                add=True))
        for cp in sB:
            cp.wait()
        return carry

    lax.fori_loop(0, H, body, 0)
    plsc.subcore_barrier()
    for k in range(-(-NRD // NS)):
        cid = sub + NS * k

        @pl.when(cid < NRD)
        def _():
            r0 = cid * RD
            pltpu.sync_copy(acc.at[pl.ds(r0, RD)], rowsA.at[pl.ds(0, RD)])
            pltpu.sync_copy(rowsA.at[pl.ds(0, RD)], out.at[chunk, pl.ds(r0, RD)])


def _sc_aggregate(x_all, idx_all, zeros):
    mesh = plsc.VectorSubcoreMesh(
        core_axis_name="c", subcore_axis_name="s", num_cores=NC,
        num_subcores=NS)
    f = pl.kernel(
        _sc_body,
        out_type=jax.ShapeDtypeStruct((NCHUNK, RN, CHUNK), jnp.bfloat16),
        mesh=mesh,
        scratch_types=[
            pltpu.VMEM_SHARED((RN + 8, CHUNK), jnp.bfloat16),
            pltpu.VMEM_SHARED((N, CHUNK), jnp.bfloat16),
            pltpu.VMEM((KB, 2, BI), jnp.int32),
            pltpu.VMEM((KB, 2, BI), jnp.int32),
            pltpu.VMEM((EB, CHUNK), jnp.bfloat16),
            pltpu.VMEM((EB, CHUNK), jnp.bfloat16),
            pltpu.SemaphoreType.DMA,
            pltpu.SemaphoreType.DMA,
            pltpu.SemaphoreType.DMA,
            pltpu.SemaphoreType.DMA,
        ],
        compiler_params=pltpu.CompilerParams(use_tc_tiling_on_sc=False),
    )
    return f(x_all, idx_all, zeros)


def _mlp1_body(x_ref, agg_ref, w_ref, b_ref, h1_ref, s1_ref, s2_ref):
    nb = pl.program_id(1)
    h = x_ref[...] + jnp.concatenate(
        [agg_ref[c, 0] for c in range(NCHUNK)], axis=1).astype(jnp.float32)
    h1 = jnp.dot(h, w_ref[0], precision=lax.Precision.DEFAULT) + b_ref[0, 0]
    h1_ref[0] = h1.astype(jnp.bfloat16)

    @pl.when(nb == 0)
    def _():
        s1_ref[...] = jnp.zeros_like(s1_ref)
        s2_ref[...] = jnp.zeros_like(s2_ref)

    s1_ref[0] += jnp.sum(h1, axis=0, keepdims=True)
    s2_ref[0] += jnp.sum(h1 * h1, axis=0, keepdims=True)


def _mlp2_body(x_ref, h1_ref, wst_ref, w2t_ref, s1_ref, s2_ref, g_ref,
               be_ref, bias_ref, o_ref):
    acc = jnp.dot(x_ref[...], wst_ref[...],
                  precision=lax.Precision.DEFAULT) + bias_ref[...]
    for r in range(R):
        mean = s1_ref[r, 0] / N
        var = s2_ref[r, 0] / N - mean * mean
        a = g_ref[r] * lax.rsqrt(var + BN_EPS)
        c = be_ref[r] - mean * a
        g = jnp.maximum(h1_ref[r].astype(jnp.float32) * a + c, 0.0)
        acc += jnp.dot(g, w2t_ref[r], precision=lax.Precision.DEFAULT)
    o_ref[...] = acc


def kernel(x, edge_index, edge_type, W_self, b_self, W1, b1, gamma, beta,
           W2, b2):
    E = edge_index.shape[1]
    row = edge_index[0].astype(jnp.int32)
    col = edge_index[1].astype(jnp.int32)
    et = edge_type.astype(jnp.int32)

    block = NS * 2 * EB
    e_pad = ((E + block - 1) // block) * block
    pad = e_pad - E
    col_p = jnp.concatenate([col, jnp.zeros((pad,), jnp.int32)])
    d_p = jnp.concatenate([et * N + row, jnp.full((pad,), RN, jnp.int32)])
    idx_all = jnp.concatenate([col_p.reshape(e_pad // BI, 1, BI),
                               d_p.reshape(e_pad // BI, 1, BI)], axis=1)
    x_all = x.astype(jnp.bfloat16).reshape(N, NCHUNK, CHUNK).transpose(
        1, 0, 2).reshape(NCHUNK * N, CHUNK)
    zeros = jnp.zeros((RD, CHUNK), jnp.bfloat16)

    aggc = _sc_aggregate(x_all, idx_all, zeros)
    agg = aggc.reshape(NCHUNK, R, N, CHUNK)

    BN = 1000
    nb_grid = N // BN
    W1T = W1.transpose(0, 2, 1)
    h1, s1, s2 = pl.pallas_call(
        _mlp1_body,
        grid=(R, nb_grid),
        in_specs=[
            pl.BlockSpec((BN, D), lambda r, nb: (nb, 0)),
            pl.BlockSpec((NCHUNK, 1, BN, CHUNK), lambda r, nb: (0, r, nb, 0)),
            pl.BlockSpec((1, D, D), lambda r, nb: (r, 0, 0)),
            pl.BlockSpec((1, 1, D), lambda r, nb: (r, 0, 0)),
        ],
        out_specs=[
            pl.BlockSpec((1, BN, D), lambda r, nb: (r, nb, 0)),
            pl.BlockSpec((1, 1, D), lambda r, nb: (r, 0, 0)),
            pl.BlockSpec((1, 1, D), lambda r, nb: (r, 0, 0)),
        ],
        out_shape=[
            jax.ShapeDtypeStruct((R, N, D), jnp.bfloat16),
            jax.ShapeDtypeStruct((R, 1, D), jnp.float32),
            jax.ShapeDtypeStruct((R, 1, D), jnp.float32),
        ],
    )(x, agg, W1T, b1.reshape(R, 1, D))

    bias_total = (b_self + jnp.sum(b2, axis=0))[None, :]

    out = pl.pallas_call(
        _mlp2_body,
        grid=(nb_grid,),
        in_specs=[
            pl.BlockSpec((BN, D), lambda nb: (nb, 0)),
            pl.BlockSpec((R, BN, D), lambda nb: (0, nb, 0)),
            pl.BlockSpec((D, D), lambda nb: (0, 0)),
            pl.BlockSpec((R, D, D), lambda nb: (0, 0, 0)),
            pl.BlockSpec((R, 1, D), lambda nb: (0, 0, 0)),
            pl.BlockSpec((R, 1, D), lambda nb: (0, 0, 0)),
            pl.BlockSpec((R, D), lambda nb: (0, 0)),
            pl.BlockSpec((R, D), lambda nb: (0, 0)),
            pl.BlockSpec((1, D), lambda nb: (0, 0)),
        ],
        out_specs=pl.BlockSpec((BN, D), lambda nb: (nb, 0)),
        out_shape=jax.ShapeDtypeStruct((N, D), jnp.float32),
    )(x, h1, W_self.T, W2.transpose(0, 2, 1), s1, s2, gamma, beta, bias_total)
    return out


# speedup vs baseline: 1.1194x; 1.0698x over previous
import functools

import jax
import jax.numpy as jnp
from jax import lax
from jax.experimental import pallas as pl
from jax.experimental.pallas import tpu as pltpu
from jax.experimental.pallas import tpu_sc as plsc

N = 10000
D = 128
R = 4
CHUNK = 64
NCHUNK = D // CHUNK
NC, NS = 2, 16
BI = 128
KB = 3
EB = KB * BI
BN_EPS = 1e-5
RN = R * N
RD = 320
NRD = -(-RN // RD)
XS = N // NS


def _sc_body(x_all, idx_all, zeros, out, acc, x_s, iA, iB, rowsA, rowsB,
             semgA, semgB, semsA, semsB):
    core = lax.axis_index("c")
    sub = lax.axis_index("s")
    n_rows = idx_all.shape[0] // NS
    rbase = sub * n_rows

    chunk = core
    pltpu.sync_copy(x_all.at[pl.ds(chunk * N + sub * XS, XS)],
                    x_s.at[pl.ds(sub * XS, XS)])
    pltpu.sync_copy(zeros, rowsA.at[pl.ds(0, RD)])
    for k in range(-(-NRD // NS)):
        cid = sub + NS * k

        @pl.when(cid < NRD)
        def _():
            pltpu.sync_copy(rowsA.at[pl.ds(0, RD)], acc.at[pl.ds(cid * RD, RD)])

    plsc.subcore_barrier()

    gA = [pltpu.make_async_copy(
        x_s.at[iA.at[j, 0]], rowsA.at[pl.ds(j * BI, BI)], semgA)
        for j in range(KB)]
    gB = [pltpu.make_async_copy(
        x_s.at[iB.at[j, 0]], rowsB.at[pl.ds(j * BI, BI)], semgB)
        for j in range(KB)]
    H = n_rows // (2 * KB)

    pltpu.sync_copy(idx_all.at[pl.ds(rbase, KB)], iA)
    for cp in gA:
        cp.start()

    def body(h, carry):
        pltpu.sync_copy(idx_all.at[pl.ds(rbase + (2 * h + 1) * KB, KB)], iB)
        for cp in gB:
            cp.start()
        sA = []
        for j in range(KB):
            gA[j].wait()
            sA.append(pltpu.async_copy(
                rowsA.at[pl.ds(j * BI, BI)], acc.at[iA.at[j, 1]], semsA,
                add=True))
        for cp in sA:
            cp.wait()

        @pl.when(h < H - 1)
        def _():
            pltpu.sync_copy(
                idx_all.at[pl.ds(rbase + (2 * h + 2) * KB, KB)], iA)
            for cp in gA:
                cp.start()

        sB = []
        for j in range(KB):
            gB[j].wait()
            sB.append(pltpu.async_copy(
                rowsB.at[pl.ds(j * BI, BI)], acc.at[iB.at[j, 1]], semsB,
                add=True))
        for cp in sB:
            cp.wait()
        return carry

    lax.fori_loop(0, H, body, 0)
    plsc.subcore_barrier()
    for k in range(-(-NRD // NS)):
        cid = sub + NS * k

        @pl.when(cid < NRD)
        def _():
            r0 = cid * RD
            pltpu.sync_copy(acc.at[pl.ds(r0, RD)], rowsA.at[pl.ds(0, RD)])
            pltpu.sync_copy(rowsA.at[pl.ds(0, RD)], out.at[chunk, pl.ds(r0, RD)])


def _sc_aggregate(x_all, idx_all, zeros):
    mesh = plsc.VectorSubcoreMesh(
        core_axis_name="c", subcore_axis_name="s", num_cores=NC,
        num_subcores=NS)
    f = pl.kernel(
        _sc_body,
        out_type=jax.ShapeDtypeStruct((NCHUNK, RN, CHUNK), jnp.bfloat16),
        mesh=mesh,
        scratch_types=[
            pltpu.VMEM_SHARED((RN + 8, CHUNK), jnp.bfloat16),
            pltpu.VMEM_SHARED((N, CHUNK), jnp.bfloat16),
            pltpu.VMEM((KB, 2, BI), jnp.int32),
            pltpu.VMEM((KB, 2, BI), jnp.int32),
            pltpu.VMEM((EB, CHUNK), jnp.bfloat16),
            pltpu.VMEM((EB, CHUNK), jnp.bfloat16),
            pltpu.SemaphoreType.DMA,
            pltpu.SemaphoreType.DMA,
            pltpu.SemaphoreType.DMA,
            pltpu.SemaphoreType.DMA,
        ],
        compiler_params=pltpu.CompilerParams(use_tc_tiling_on_sc=False),
    )
    return f(x_all, idx_all, zeros)


def _mlp1_body(x_ref, agg_ref, w_ref, b_ref, h1_ref, s1_ref, s2_ref):
    nb = pl.program_id(1)
    h = x_ref[...] + jnp.concatenate(
        [agg_ref[c, 0] for c in range(NCHUNK)], axis=1).astype(jnp.float32)
    h1 = jnp.dot(h, w_ref[0], precision=lax.Precision.DEFAULT) + b_ref[0, 0]
    h1_ref[0] = h1.astype(jnp.bfloat16)

    @pl.when(nb == 0)
    def _():
        s1_ref[...] = jnp.zeros_like(s1_ref)
        s2_ref[...] = jnp.zeros_like(s2_ref)

    s1_ref[0] += jnp.sum(h1, axis=0, keepdims=True)
    s2_ref[0] += jnp.sum(h1 * h1, axis=0, keepdims=True)


def _mlp2_body(x_ref, h1_ref, wst_ref, w2t_ref, s1_ref, s2_ref, g_ref,
               be_ref, bias_ref, o_ref):
    acc = jnp.dot(x_ref[...], wst_ref[...],
                  precision=lax.Precision.DEFAULT) + bias_ref[...]
    for r in range(R):
        mean = s1_ref[r, 0] / N
        var = s2_ref[r, 0] / N - mean * mean
        a = g_ref[r] * lax.rsqrt(var + BN_EPS)
        c = be_ref[r] - mean * a
        g = jnp.maximum(h1_ref[r].astype(jnp.float32) * a + c, 0.0)
        acc += jnp.dot(g, w2t_ref[r], precision=lax.Precision.DEFAULT)
    o_ref[...] = acc


def kernel(x, edge_index, edge_type, W_self, b_self, W1, b1, gamma, beta,
           W2, b2):
    E = edge_index.shape[1]
    row = edge_index[0].astype(jnp.int32)
    col = edge_index[1].astype(jnp.int32)
    et = edge_type.astype(jnp.int32)

    block = NS * 2 * EB
    e_pad = ((E + block - 1) // block) * block
    pad = e_pad - E
    col_p = jnp.concatenate([col, jnp.zeros((pad,), jnp.int32)])
    d_p = jnp.concatenate([et * N + row, jnp.full((pad,), RN, jnp.int32)])
    idx_all = jnp.concatenate([col_p.reshape(e_pad // BI, 1, BI),
                               d_p.reshape(e_pad // BI, 1, BI)], axis=1)
    x_all = x.astype(jnp.bfloat16).reshape(N, NCHUNK, CHUNK).transpose(
        1, 0, 2).reshape(NCHUNK * N, CHUNK)
    zeros = jnp.zeros((RD, CHUNK), jnp.bfloat16)

    aggc = _sc_aggregate(x_all, idx_all, zeros)
    agg = aggc.reshape(NCHUNK, R, N, CHUNK)

    BN = 2000
    nb_grid = N // BN
    W1T = W1.transpose(0, 2, 1)
    h1, s1, s2 = pl.pallas_call(
        _mlp1_body,
        grid=(R, nb_grid),
        in_specs=[
            pl.BlockSpec((BN, D), lambda r, nb: (nb, 0)),
            pl.BlockSpec((NCHUNK, 1, BN, CHUNK), lambda r, nb: (0, r, nb, 0)),
            pl.BlockSpec((1, D, D), lambda r, nb: (r, 0, 0)),
            pl.BlockSpec((1, 1, D), lambda r, nb: (r, 0, 0)),
        ],
        out_specs=[
            pl.BlockSpec((1, BN, D), lambda r, nb: (r, nb, 0)),
            pl.BlockSpec((1, 1, D), lambda r, nb: (r, 0, 0)),
            pl.BlockSpec((1, 1, D), lambda r, nb: (r, 0, 0)),
        ],
        out_shape=[
            jax.ShapeDtypeStruct((R, N, D), jnp.bfloat16),
            jax.ShapeDtypeStruct((R, 1, D), jnp.float32),
            jax.ShapeDtypeStruct((R, 1, D), jnp.float32),
        ],
    )(x, agg, W1T, b1.reshape(R, 1, D))

    bias_total = (b_self + jnp.sum(b2, axis=0))[None, :]

    out = pl.pallas_call(
        _mlp2_body,
        grid=(nb_grid,),
        in_specs=[
            pl.BlockSpec((BN, D), lambda nb: (nb, 0)),
            pl.BlockSpec((R, BN, D), lambda nb: (0, nb, 0)),
            pl.BlockSpec((D, D), lambda nb: (0, 0)),
            pl.BlockSpec((R, D, D), lambda nb: (0, 0, 0)),
            pl.BlockSpec((R, 1, D), lambda nb: (0, 0, 0)),
            pl.BlockSpec((R, 1, D), lambda nb: (0, 0, 0)),
            pl.BlockSpec((R, D), lambda nb: (0, 0)),
            pl.BlockSpec((R, D), lambda nb: (0, 0)),
            pl.BlockSpec((1, D), lambda nb: (0, 0)),
        ],
        out_specs=pl.BlockSpec((BN, D), lambda nb: (nb, 0)),
        out_shape=jax.ShapeDtypeStruct((N, D), jnp.float32),
    )(x, h1, W_self.T, W2.transpose(0, 2, 1), s1, s2, gamma, beta, bias_total)
    return out
